# baseline (speedup 1.0000x reference)
import jax
import jax.numpy as jnp
from jax.experimental import pallas as pl
from jax.experimental.pallas import tpu as pltpu

_BATCH = 64
_VOCAB = 100000
_CHUNK = 6400
_GRID = (_VOCAB + _CHUNK - 1) // _CHUNK

_K0 = 0
_K1 = 42
_KS2 = 0x1BD11BDA ^ _K0 ^ _K1


def _rotl(x, r):
    return jax.lax.shift_left(x, jnp.uint32(r)) | jax.lax.shift_right_logical(
        x, jnp.uint32(32 - r)
    )


def _threefry2x32(x0, x1):
    ks = (jnp.uint32(_K0), jnp.uint32(_K1), jnp.uint32(_KS2))
    x0 = x0 + ks[0]
    x1 = x1 + ks[1]
    rot_even = (13, 15, 26, 6)
    rot_odd = (17, 29, 16, 24)
    inject = ((ks[1], ks[2]), (ks[2], ks[0]), (ks[0], ks[1]),
              (ks[1], ks[2]), (ks[2], ks[0]))
    for i, (ka, kb) in enumerate(inject):
        for r in rot_even if i % 2 == 0 else rot_odd:
            x0 = x0 + x1
            x1 = _rotl(x1, r)
            x1 = x1 ^ x0
        x0 = x0 + ka
        x1 = x1 + kb + jnp.uint32(i + 1)
    return x0 ^ x1


def _gumbel(lin_idx_u32):
    bits = _threefry2x32(jnp.zeros_like(lin_idx_u32), lin_idx_u32)
    mant = jax.lax.shift_right_logical(bits, jnp.uint32(9)) | jnp.uint32(0x3F800000)
    f = jax.lax.bitcast_convert_type(mant, jnp.float32) - jnp.float32(1.0)
    span = jnp.float32(1.0 - 1e-20)
    u = jnp.maximum(jnp.float32(1e-20), f * span + jnp.float32(1e-20))
    return -jnp.log(-jnp.log(u))


def _body(logits_ref, mask_ref, ids_ref, masked_ref, rmax_ref, ridx_ref):
    c = pl.program_id(0)
    masked = logits_ref[...] + mask_ref[...]
    masked_ref[...] = masked

    col = jax.lax.broadcasted_iota(jnp.int32, (_BATCH, _CHUNK), 1) + c * _CHUNK
    row = jax.lax.broadcasted_iota(jnp.int32, (_BATCH, _CHUNK), 0)
    lin = (row * _VOCAB + col).astype(jnp.uint32)
    score = masked + _gumbel(lin)
    score = jnp.where(col < _VOCAB, score, -jnp.inf)

    cmax = jnp.max(score, axis=1, keepdims=True)
    cidx = jnp.min(
        jnp.where(score == cmax, col, jnp.int32(_VOCAB)), axis=1, keepdims=True
    )

    @pl.when(c == 0)
    def _():
        rmax_ref[...] = cmax
        ridx_ref[...] = cidx

    @pl.when(c > 0)
    def _():
        upd = cmax > rmax_ref[...]
        rmax_ref[...] = jnp.where(upd, cmax, rmax_ref[...])
        ridx_ref[...] = jnp.where(upd, cidx, ridx_ref[...])

    @pl.when(c == _GRID - 1)
    def _():
        ids_ref[...] = ridx_ref[...]


def _run(logits, mask2d, interpret=False):
    return pl.pallas_call(
        _body,
        grid=(_GRID,),
        in_specs=[
            pl.BlockSpec((_BATCH, _CHUNK), lambda c: (0, c)),
            pl.BlockSpec((1, _CHUNK), lambda c: (0, c)),
        ],
        out_specs=[
            pl.BlockSpec((_BATCH, 1), lambda c: (0, 0)),
            pl.BlockSpec((_BATCH, _CHUNK), lambda c: (0, c)),
        ],
        out_shape=[
            jax.ShapeDtypeStruct((_BATCH, 1), jnp.int32),
            jax.ShapeDtypeStruct((_BATCH, _VOCAB), jnp.float32),
        ],
        scratch_shapes=[
            pltpu.VMEM((_BATCH, 1), jnp.float32),
            pltpu.VMEM((_BATCH, 1), jnp.int32),
        ],
        interpret=interpret,
    )(logits, mask2d)


def kernel(logits, prediction_mask):
    ids, masked = _run(logits, prediction_mask.reshape(1, _VOCAB))
    return ids.reshape(_BATCH), masked


# speedup vs baseline: 1.1730x; 1.1730x over previous
import jax
import jax.numpy as jnp
from jax.experimental import pallas as pl
from jax.experimental.pallas import tpu as pltpu

_BATCH = 64
_VOCAB = 100000
_CHUNK = 6400
_GRID = (_VOCAB + _CHUNK - 1) // _CHUNK
_TILE = 256
_NTILE = _CHUNK // _TILE

_K1 = 42
_KS2 = 0x1BD11BDA ^ 42


def _rotl(x, r):
    return jax.lax.shift_left(x, jnp.uint32(r)) | jax.lax.shift_right_logical(
        x, jnp.uint32(32 - r)
    )


def _threefry2x32(x1):
    ks1 = jnp.uint32(_K1)
    ks2 = jnp.uint32(_KS2)
    x0 = jnp.zeros_like(x1)
    x1 = x1 + ks1
    rot_even = (13, 15, 26, 6)
    rot_odd = (17, 29, 16, 24)
    inject = (
        (ks1, ks2 + jnp.uint32(1)),
        (ks2, jnp.uint32(2)),
        (jnp.uint32(0), ks1 + jnp.uint32(3)),
        (ks1, ks2 + jnp.uint32(4)),
        (ks2, jnp.uint32(5)),
    )
    for i, (ka, kb) in enumerate(inject):
        for r in rot_even if i % 2 == 0 else rot_odd:
            x0 = x0 + x1
            x1 = _rotl(x1, r)
            x1 = x1 ^ x0
        if i != 2:
            x0 = x0 + ka
        x1 = x1 + kb
    return x0 ^ x1


def _gumbel(lin_idx_u32):
    bits = _threefry2x32(lin_idx_u32)
    mant = jax.lax.shift_right_logical(bits, jnp.uint32(9)) | jnp.uint32(0x3F800000)
    f = jax.lax.bitcast_convert_type(mant, jnp.float32) - jnp.float32(1.0)
    span = jnp.float32(1.0 - 1e-20)
    u = jnp.maximum(jnp.float32(1e-20), f * span + jnp.float32(1e-20))
    return -jnp.log(-jnp.log(u))


def _body(logits_ref, mask_ref, ids_ref, masked_ref, score_ref, rmax_ref, ridx_ref):
    c = pl.program_id(0)

    base = jax.lax.broadcasted_iota(jnp.int32, (_BATCH, _TILE), 0) * _VOCAB + (
        jax.lax.broadcasted_iota(jnp.int32, (_BATCH, _TILE), 1) + c * _CHUNK
    )

    def tile_step(t, _):
        sl = pl.ds(t * _TILE, _TILE)
        masked = logits_ref[:, sl] + mask_ref[:, sl]
        masked_ref[:, sl] = masked
        lin = (base + t * _TILE).astype(jnp.uint32)
        score_ref[:, sl] = masked + _gumbel(lin)
        return _

    jax.lax.fori_loop(0, _NTILE, tile_step, 0, unroll=False)

    col = jax.lax.broadcasted_iota(jnp.int32, (_BATCH, _CHUNK), 1) + c * _CHUNK
    score = jnp.where(col < _VOCAB, score_ref[...], -jnp.inf)
    cmax = jnp.max(score, axis=1, keepdims=True)
    cidx = jnp.min(
        jnp.where(score == cmax, col, jnp.int32(_VOCAB)), axis=1, keepdims=True
    )

    @pl.when(c == 0)
    def _():
        rmax_ref[...] = cmax
        ridx_ref[...] = cidx

    @pl.when(c > 0)
    def _():
        upd = cmax > rmax_ref[...]
        rmax_ref[...] = jnp.where(upd, cmax, rmax_ref[...])
        ridx_ref[...] = jnp.where(upd, cidx, ridx_ref[...])

    @pl.when(c == _GRID - 1)
    def _():
        ids_ref[...] = ridx_ref[...]


def _run(logits, mask2d, interpret=False):
    return pl.pallas_call(
        _body,
        grid=(_GRID,),
        in_specs=[
            pl.BlockSpec((_BATCH, _CHUNK), lambda c: (0, c)),
            pl.BlockSpec((1, _CHUNK), lambda c: (0, c)),
        ],
        out_specs=[
            pl.BlockSpec((_BATCH, 1), lambda c: (0, 0)),
            pl.BlockSpec((_BATCH, _CHUNK), lambda c: (0, c)),
        ],
        out_shape=[
            jax.ShapeDtypeStruct((_BATCH, 1), jnp.int32),
            jax.ShapeDtypeStruct((_BATCH, _VOCAB), jnp.float32),
        ],
        scratch_shapes=[
            pltpu.VMEM((_BATCH, _CHUNK), jnp.float32),
            pltpu.VMEM((_BATCH, 1), jnp.float32),
            pltpu.VMEM((_BATCH, 1), jnp.int32),
        ],
        interpret=interpret,
    )(logits, mask2d)


def kernel(logits, prediction_mask):
    ids, masked = _run(logits, prediction_mask.reshape(1, _VOCAB))
    return ids.reshape(_BATCH), masked


# speedup vs baseline: 1.2947x; 1.1037x over previous
import jax
import jax.numpy as jnp
from jax.experimental import pallas as pl
from jax.experimental.pallas import tpu as pltpu

_BATCH = 64
_VOCAB = 100000
_CHUNK = 6400
_GRID = (_VOCAB + _CHUNK - 1) // _CHUNK
_TILE = 512
_NTILE = _CHUNK // _TILE

_K1 = 42
_KS2 = 0x1BD11BDA ^ 42


def _rotl(x, r):
    return jax.lax.shift_left(x, jnp.uint32(r)) | jax.lax.shift_right_logical(
        x, jnp.uint32(32 - r)
    )


def _threefry2x32(x1):
    ks1 = jnp.uint32(_K1)
    ks2 = jnp.uint32(_KS2)
    x0 = jnp.zeros_like(x1)
    x1 = x1 + ks1
    rot_even = (13, 15, 26, 6)
    rot_odd = (17, 29, 16, 24)
    inject = (
        (ks1, ks2 + jnp.uint32(1)),
        (ks2, jnp.uint32(2)),
        (jnp.uint32(0), ks1 + jnp.uint32(3)),
        (ks1, ks2 + jnp.uint32(4)),
        (ks2, jnp.uint32(5)),
    )
    for i, (ka, kb) in enumerate(inject):
        for r in rot_even if i % 2 == 0 else rot_odd:
            x0 = x0 + x1
            x1 = _rotl(x1, r)
            x1 = x1 ^ x0
        if i != 2:
            x0 = x0 + ka
        x1 = x1 + kb
    return x0 ^ x1


def _gumbel(lin_idx_u32):
    bits = _threefry2x32(lin_idx_u32)
    mant = jax.lax.shift_right_logical(bits, jnp.uint32(9)) | jnp.uint32(0x3F800000)
    f = jax.lax.bitcast_convert_type(mant, jnp.float32) - jnp.float32(1.0)
    span = jnp.float32(1.0 - 1e-20)
    u = jnp.maximum(jnp.float32(1e-20), f * span + jnp.float32(1e-20))
    return -jnp.log(-jnp.log(u))


def _body(logits_ref, mask_ref, ids_ref, masked_ref, score_ref, rmax_ref, ridx_ref):
    c = pl.program_id(0)

    base = jax.lax.broadcasted_iota(jnp.int32, (_BATCH, _TILE), 0) * _VOCAB + (
        jax.lax.broadcasted_iota(jnp.int32, (_BATCH, _TILE), 1) + c * _CHUNK
    )

    def tile_step(t, _):
        sl = pl.ds(t * _TILE, _TILE)
        masked = logits_ref[:, sl] + mask_ref[:, sl]
        masked_ref[:, sl] = masked
        lin = (base + t * _TILE).astype(jnp.uint32)
        score_ref[:, sl] = masked + _gumbel(lin)
        return _

    jax.lax.fori_loop(0, _NTILE, tile_step, 0, unroll=False)

    col = jax.lax.broadcasted_iota(jnp.int32, (_BATCH, _CHUNK), 1) + c * _CHUNK
    score = jnp.where(col < _VOCAB, score_ref[...], -jnp.inf)
    cmax = jnp.max(score, axis=1, keepdims=True)
    cidx = jnp.min(
        jnp.where(score == cmax, col, jnp.int32(_VOCAB)), axis=1, keepdims=True
    )

    @pl.when(c == 0)
    def _():
        rmax_ref[...] = cmax
        ridx_ref[...] = cidx

    @pl.when(c > 0)
    def _():
        upd = cmax > rmax_ref[...]
        rmax_ref[...] = jnp.where(upd, cmax, rmax_ref[...])
        ridx_ref[...] = jnp.where(upd, cidx, ridx_ref[...])

    @pl.when(c == _GRID - 1)
    def _():
        ids_ref[...] = ridx_ref[...]


def _run(logits, mask2d, interpret=False):
    return pl.pallas_call(
        _body,
        grid=(_GRID,),
        in_specs=[
            pl.BlockSpec((_BATCH, _CHUNK), lambda c: (0, c)),
            pl.BlockSpec((1, _CHUNK), lambda c: (0, c)),
        ],
        out_specs=[
            pl.BlockSpec((_BATCH, 1), lambda c: (0, 0)),
            pl.BlockSpec((_BATCH, _CHUNK), lambda c: (0, c)),
        ],
        out_shape=[
            jax.ShapeDtypeStruct((_BATCH, 1), jnp.int32),
            jax.ShapeDtypeStruct((_BATCH, _VOCAB), jnp.float32),
        ],
        scratch_shapes=[
            pltpu.VMEM((_BATCH, _CHUNK), jnp.float32),
            pltpu.VMEM((_BATCH, 1), jnp.float32),
            pltpu.VMEM((_BATCH, 1), jnp.int32),
        ],
        interpret=interpret,
    )(logits, mask2d)


def kernel(logits, prediction_mask):
    ids, masked = _run(logits, prediction_mask.reshape(1, _VOCAB))
    return ids.reshape(_BATCH), masked
